# baseline (device time: 191012 ns/iter reference)
import jax
import jax.numpy as jnp
from jax import lax
from jax.experimental import pallas as pl
from jax.experimental.pallas import tpu as pltpu

N_DEV = 4


def _allreduce_body(x_ref, out_ref, comm_ref, send_sems, recv_sems):
    my_pos = lax.axis_index("i")
    left = (my_pos - 1) % N_DEV
    right = (my_pos + 1) % N_DEV

    barrier_sem = pltpu.get_barrier_semaphore()
    for nbr in [left, right]:
        pl.semaphore_signal(
            barrier_sem, inc=1,
            device_id=(nbr,), device_id_type=pl.DeviceIdType.MESH,
        )
    pl.semaphore_wait(barrier_sem, 2)

    out_ref[:, :] = x_ref[:, :]
    comm_ref[0, :, :] = x_ref[:, :]

    for h in range(N_DEV - 1):
        send_slot = h % 2
        recv_slot = (h + 1) % 2
        rdma = pltpu.make_async_remote_copy(
            src_ref=comm_ref.at[send_slot],
            dst_ref=comm_ref.at[recv_slot],
            send_sem=send_sems.at[send_slot],
            recv_sem=recv_sems.at[recv_slot],
            device_id=(right,),
            device_id_type=pl.DeviceIdType.MESH,
        )
        rdma.start()
        rdma.wait()
        out_ref[:, :] = out_ref[:, :] + comm_ref[recv_slot, :, :]


def kernel(ids, E):
    v_per, d = E.shape
    t = ids.shape[0]
    my_pos = lax.axis_index("i")

    local = ids - my_pos * v_per
    mask = (local >= 0) & (local < v_per)
    idx = jnp.where(mask, local, 0)
    partial = jnp.where(mask[:, None], jnp.take(E, idx, axis=0), 0.0)
    partial = partial.astype(jnp.float32)

    return pl.pallas_call(
        _allreduce_body,
        out_shape=jax.ShapeDtypeStruct((t, d), jnp.float32),
        in_specs=[pl.BlockSpec(memory_space=pltpu.VMEM)],
        out_specs=pl.BlockSpec(memory_space=pltpu.VMEM),
        scratch_shapes=[
            pltpu.VMEM((2, t, d), jnp.float32),
            pltpu.SemaphoreType.DMA((2,)),
            pltpu.SemaphoreType.DMA((2,)),
        ],
        compiler_params=pltpu.CompilerParams(collective_id=0),
    )(partial)


# device time: 93708 ns/iter; 2.0384x vs baseline; 2.0384x over previous
import jax
import jax.numpy as jnp
from jax import lax
from jax.experimental import pallas as pl
from jax.experimental.pallas import tpu as pltpu

N_DEV = 4
N_STEP = 2 * (N_DEV - 1)


def _ar_body(x_ref, out_ref, cw_buf, ccw_buf,
             cw_ssem, cw_rsem, ccw_ssem, ccw_rsem):
    t, d = x_ref.shape
    half = t // 2
    chunk = half // N_DEV

    my = lax.axis_index("i")
    left = (my - 1) % N_DEV
    right = (my + 1) % N_DEV

    def cw_rows(c):
        return pl.ds(c * chunk, chunk)

    def ccw_rows(c):
        return pl.ds(half + c * chunk, chunk)

    barrier_sem = pltpu.get_barrier_semaphore()
    for nbr in [left, right]:
        pl.semaphore_signal(
            barrier_sem, inc=1,
            device_id=(nbr,), device_id_type=pl.DeviceIdType.MESH,
        )
    pl.semaphore_wait(barrier_sem, 2)

    out_ref[:, :] = x_ref[:, :]

    for s in range(N_DEV - 1):
        cw_sc = (my - s) % N_DEV
        cw_rc = (my - s - 1) % N_DEV
        ccw_sc = (my + s) % N_DEV
        ccw_rc = (my + s + 1) % N_DEV
        cw = pltpu.make_async_remote_copy(
            src_ref=out_ref.at[cw_rows(cw_sc), :],
            dst_ref=cw_buf.at[s],
            send_sem=cw_ssem.at[s],
            recv_sem=cw_rsem.at[s],
            device_id=(right,),
            device_id_type=pl.DeviceIdType.MESH,
        )
        ccw = pltpu.make_async_remote_copy(
            src_ref=out_ref.at[ccw_rows(ccw_sc), :],
            dst_ref=ccw_buf.at[s],
            send_sem=ccw_ssem.at[s],
            recv_sem=ccw_rsem.at[s],
            device_id=(left,),
            device_id_type=pl.DeviceIdType.MESH,
        )
        cw.start()
        ccw.start()
        cw.wait()
        ccw.wait()
        out_ref[cw_rows(cw_rc), :] = (
            out_ref[cw_rows(cw_rc), :] + cw_buf[s, :, :]
        )
        out_ref[ccw_rows(ccw_rc), :] = (
            out_ref[ccw_rows(ccw_rc), :] + ccw_buf[s, :, :]
        )

    for s in range(N_DEV - 1):
        k = (N_DEV - 1) + s
        cw_sc = (my + 1 - s) % N_DEV
        cw_rc = (my - s) % N_DEV
        ccw_sc = (my - 1 + s) % N_DEV
        ccw_rc = (my + s) % N_DEV
        cw = pltpu.make_async_remote_copy(
            src_ref=out_ref.at[cw_rows(cw_sc), :],
            dst_ref=out_ref.at[cw_rows(cw_sc), :],
            send_sem=cw_ssem.at[k],
            recv_sem=cw_rsem.at[k],
            device_id=(right,),
            device_id_type=pl.DeviceIdType.MESH,
        )
        ccw = pltpu.make_async_remote_copy(
            src_ref=out_ref.at[ccw_rows(ccw_sc), :],
            dst_ref=out_ref.at[ccw_rows(ccw_sc), :],
            send_sem=ccw_ssem.at[k],
            recv_sem=ccw_rsem.at[k],
            device_id=(left,),
            device_id_type=pl.DeviceIdType.MESH,
        )
        cw.start()
        ccw.start()
        cw.wait()
        ccw.wait()


def kernel(ids, E):
    v_per, d = E.shape
    t = ids.shape[0]
    my_pos = lax.axis_index("i")

    local = ids - my_pos * v_per
    mask = (local >= 0) & (local < v_per)
    idx = jnp.where(mask, local, 0)
    partial = jnp.where(mask[:, None], jnp.take(E, idx, axis=0), 0.0)
    partial = partial.astype(jnp.float32)

    chunk = t // (2 * N_DEV)
    return pl.pallas_call(
        _ar_body,
        out_shape=jax.ShapeDtypeStruct((t, d), jnp.float32),
        in_specs=[pl.BlockSpec(memory_space=pltpu.VMEM)],
        out_specs=pl.BlockSpec(memory_space=pltpu.VMEM),
        scratch_shapes=[
            pltpu.VMEM((N_DEV - 1, chunk, d), jnp.float32),
            pltpu.VMEM((N_DEV - 1, chunk, d), jnp.float32),
            pltpu.SemaphoreType.DMA((N_STEP,)),
            pltpu.SemaphoreType.DMA((N_STEP,)),
            pltpu.SemaphoreType.DMA((N_STEP,)),
            pltpu.SemaphoreType.DMA((N_STEP,)),
        ],
        compiler_params=pltpu.CompilerParams(collective_id=0),
    )(partial)
